# baseline (speedup 1.0000x reference)
import functools

import jax
import jax.numpy as jnp
from jax import lax
from jax.experimental import pallas as pl
from jax.experimental.pallas import tpu as pltpu
from jax.experimental.pallas import tpu_sc as plsc

_BT = 256
_BR = 512
_NW = 32
_CH = 16


def _router_rank_body(x_ref, rw_ref, w_ref, idx_ref, rank_ref, counts_ref,
                      xbf_ref, lt_ref, carry_ref):
    c = pl.program_id(0)
    bt = x_ref.shape[0]
    E = rw_ref.shape[0]

    @pl.when(c == 0)
    def _init():
        ii = lax.broadcasted_iota(jnp.int32, (bt, bt), 0)
        jj = lax.broadcasted_iota(jnp.int32, (bt, bt), 1)
        lt_ref[...] = (jj < ii).astype(jnp.float32)
        carry_ref[...] = jnp.zeros_like(carry_ref)

    x = x_ref[...]
    xbf_ref[...] = x.astype(jnp.bfloat16)
    logits = lax.dot_general(
        x, rw_ref[...], (((1,), (1,)), ((), ())),
        preferred_element_type=jnp.float32,
    )
    s = jax.nn.sigmoid(logits)
    eio = lax.broadcasted_iota(jnp.int32, (bt, E), 1)
    m1 = jnp.max(s, axis=1, keepdims=True)
    i1 = jnp.min(jnp.where(s == m1, eio, E), axis=1, keepdims=True)
    s2 = jnp.where(eio == i1, jnp.float32(-1.0), s)
    m2 = jnp.max(s2, axis=1, keepdims=True)
    i2 = jnp.min(jnp.where(s2 == m2, eio, E), axis=1, keepdims=True)
    denom = m1 + m2 + jnp.float32(1e-20)
    w_ref[...] = jnp.concatenate([m1, m2], axis=1) / denom
    idx_ref[...] = jnp.concatenate([i1, i2], axis=1)

    oh0 = (eio == i1).astype(jnp.float32)
    oh1 = (eio == i2).astype(jnp.float32)
    both = oh0 + oh1
    pfx = lax.dot_general(
        lt_ref[...], both, (((1,), (0,)), ((), ())),
        preferred_element_type=jnp.float32,
    )
    base = carry_ref[0:1, :]
    r = pfx + base
    rank0 = jnp.sum(oh0 * r, axis=1, keepdims=True)
    rank1 = jnp.sum(oh1 * r, axis=1, keepdims=True)
    rank_ref[...] = jnp.concatenate([rank0, rank1], axis=1).astype(jnp.int32)
    newc = base + jnp.sum(both, axis=0, keepdims=True)
    carry_ref[0:1, :] = newc
    counts_ref[...] = newc.astype(jnp.int32)


def _sc_dst(idx_v, rank_v, pc_v, fo):
    ii = lax.iota(jnp.int32, 16)
    e0 = plsc.load_gather(idx_v, [fo + 2 * ii])
    e1 = plsc.load_gather(idx_v, [fo + 2 * ii + 1])
    r0 = plsc.load_gather(rank_v, [fo + 2 * ii])
    r1 = plsc.load_gather(rank_v, [fo + 2 * ii + 1])
    dst0 = plsc.load_gather(pc_v, [e0]) + r0
    dst1 = plsc.load_gather(pc_v, [e1]) + r1
    return dst0, dst1


def _dispatch_body(xbf_hbm, idx_hbm, rank_hbm, pc_hbm, xs_hbm,
                   idx_v, rank_v, pc_v, rows_v, sem):
    wid = lax.axis_index("s") * 2 + lax.axis_index("c")
    tpw = xbf_hbm.shape[0] // _NW
    base_tok = wid * tpw
    pltpu.sync_copy(pc_hbm, pc_v)
    pltpu.sync_copy(idx_hbm.at[pl.ds(base_tok * 2, tpw * 2)], idx_v)
    pltpu.sync_copy(rank_hbm.at[pl.ds(base_tok * 2, tpw * 2)], rank_v)
    for ci in range(tpw // _CH):
        tb = ci * _CH
        pltpu.sync_copy(xbf_hbm.at[pl.ds(base_tok + tb, _CH)], rows_v)
        dst0, dst1 = _sc_dst(idx_v, rank_v, pc_v, tb * 2)
        a = pltpu.async_copy(rows_v, xs_hbm.at[dst0], sem)
        b = pltpu.async_copy(rows_v, xs_hbm.at[dst1], sem)
        a.wait()
        b.wait()


def _combine_body(y_hbm, idx_hbm, rank_hbm, pc_hbm, y0_hbm, y1_hbm,
                  idx_v, rank_v, pc_v, r0_v, r1_v, sem):
    wid = lax.axis_index("s") * 2 + lax.axis_index("c")
    tpw = y0_hbm.shape[0] // _NW
    base_tok = wid * tpw
    pltpu.sync_copy(pc_hbm, pc_v)
    pltpu.sync_copy(idx_hbm.at[pl.ds(base_tok * 2, tpw * 2)], idx_v)
    pltpu.sync_copy(rank_hbm.at[pl.ds(base_tok * 2, tpw * 2)], rank_v)
    for ci in range(tpw // _CH):
        tb = ci * _CH
        dst0, dst1 = _sc_dst(idx_v, rank_v, pc_v, tb * 2)
        g0 = pltpu.async_copy(y_hbm.at[dst0], r0_v, sem)
        g1 = pltpu.async_copy(y_hbm.at[dst1], r1_v, sem)
        g0.wait()
        g1.wait()
        s0 = pltpu.async_copy(r0_v, y0_hbm.at[pl.ds(base_tok + tb, _CH)], sem)
        s1 = pltpu.async_copy(r1_v, y1_hbm.at[pl.ds(base_tok + tb, _CH)], sem)
        s0.wait()
        s1.wait()


def _moe_body(be_ref, ba_ref, xs_ref, g_ref, u_ref, d_ref, y_ref):
    b = pl.program_id(0)

    @pl.when(ba_ref[b] == 1)
    def _():
        xb = xs_ref[...]
        t1 = lax.dot_general(
            xb, g_ref[0], (((1,), (1,)), ((), ())),
            preferred_element_type=jnp.float32,
        )
        t2 = lax.dot_general(
            xb, u_ref[0], (((1,), (1,)), ((), ())),
            preferred_element_type=jnp.float32,
        )
        h = t1 * jax.nn.sigmoid(t1) * t2
        o = lax.dot_general(
            h, d_ref[0], (((1,), (1,)), ((), ())),
            preferred_element_type=jnp.float32,
        )
        y_ref[...] = o.astype(y_ref.dtype)

    @pl.when(ba_ref[b] == 0)
    def _():
        y_ref[...] = jnp.zeros_like(y_ref)


@functools.partial(jax.jit, static_argnames=())
def kernel(hidden_states, router_w, gate_w, up_w, down_w):
    orig_shape = hidden_states.shape
    H = orig_shape[-1]
    x = hidden_states.reshape(-1, H)
    T = x.shape[0]
    E, F, _ = gate_w.shape
    K = 2
    N = T * K
    n_pad = N + E * _BT
    nb = n_pad // _BT

    w2, idx2, rank2, counts, x_bf = pl.pallas_call(
        _router_rank_body,
        grid=(T // _BR,),
        in_specs=[
            pl.BlockSpec((_BR, H), lambda i: (i, 0)),
            pl.BlockSpec((E, H), lambda i: (0, 0)),
        ],
        out_specs=[
            pl.BlockSpec((_BR, K), lambda i: (i, 0)),
            pl.BlockSpec((_BR, K), lambda i: (i, 0)),
            pl.BlockSpec((_BR, K), lambda i: (i, 0)),
            pl.BlockSpec((1, E), lambda i: (0, 0)),
            pl.BlockSpec((_BR, H), lambda i: (i, 0)),
        ],
        out_shape=[
            jax.ShapeDtypeStruct((T, K), jnp.float32),
            jax.ShapeDtypeStruct((T, K), jnp.int32),
            jax.ShapeDtypeStruct((T, K), jnp.int32),
            jax.ShapeDtypeStruct((1, E), jnp.int32),
            jax.ShapeDtypeStruct((T, H), jnp.bfloat16),
        ],
        scratch_shapes=[
            pltpu.VMEM((_BR, _BR), jnp.float32),
            pltpu.VMEM((8, E), jnp.float32),
        ],
    )(x, router_w)

    cnt = counts[0]
    padded = ((cnt + _BT - 1) // _BT) * _BT
    pad_cum = jnp.concatenate(
        [jnp.zeros((1,), jnp.int32), jnp.cumsum(padded)[:-1].astype(jnp.int32)]
    )
    pad_cum16 = jnp.concatenate([pad_cum, jnp.zeros((16 - E,), jnp.int32)])

    block_starts = jnp.arange(nb, dtype=jnp.int32) * _BT
    cmp = (block_starts[:, None] >= pad_cum[None, :]).astype(jnp.int32)
    block_expert = jnp.sum(cmp, axis=1) - 1
    be_oh = block_expert[:, None] == jnp.arange(E)[None, :]
    pc_sel = jnp.sum(be_oh * pad_cum[None, :], axis=1).astype(jnp.int32)
    c_sel = jnp.sum(be_oh * cnt[None, :], axis=1).astype(jnp.int32)
    block_active = (block_starts - pc_sel < c_sel).astype(jnp.int32)

    idx_flat = idx2.reshape(-1)
    rank_flat = rank2.reshape(-1)

    Hw = H // 2
    mesh = plsc.VectorSubcoreMesh(core_axis_name="c", subcore_axis_name="s")
    x_i32 = lax.bitcast_convert_type(
        x_bf.reshape(T, Hw, 2), jnp.int32
    ).reshape(T, 8, Hw // 8)
    sc_params = pltpu.CompilerParams(needs_layout_passes=False)
    dispatch = functools.partial(
        pl.kernel,
        mesh=mesh,
        compiler_params=sc_params,
        out_type=jax.ShapeDtypeStruct((n_pad, 8, Hw // 8), jnp.int32),
        scratch_types=[
            pltpu.VMEM((2 * T // _NW,), jnp.int32),
            pltpu.VMEM((2 * T // _NW,), jnp.int32),
            pltpu.VMEM((16,), jnp.int32),
            pltpu.VMEM((_CH, 8, Hw // 8), jnp.int32),
            pltpu.SemaphoreType.DMA,
        ],
    )(_dispatch_body)
    xs3 = dispatch(x_i32, idx_flat, rank_flat, pad_cum16)
    xs = lax.bitcast_convert_type(
        xs3.reshape(n_pad, Hw), jnp.bfloat16
    ).reshape(n_pad, H)

    y = pl.pallas_call(
        _moe_body,
        grid_spec=pltpu.PrefetchScalarGridSpec(
            num_scalar_prefetch=2,
            grid=(nb,),
            in_specs=[
                pl.BlockSpec((_BT, H), lambda b, be, ba: (b, 0)),
                pl.BlockSpec((1, F, H), lambda b, be, ba: (be[b], 0, 0)),
                pl.BlockSpec((1, F, H), lambda b, be, ba: (be[b], 0, 0)),
                pl.BlockSpec((1, H, F), lambda b, be, ba: (be[b], 0, 0)),
            ],
            out_specs=pl.BlockSpec((_BT, H), lambda b, be, ba: (b, 0)),
        ),
        out_shape=jax.ShapeDtypeStruct((n_pad, H), jnp.bfloat16),
        compiler_params=pltpu.CompilerParams(
            dimension_semantics=("arbitrary",),
        ),
    )(block_expert, block_active, xs, gate_w, up_w, down_w)

    y3 = lax.bitcast_convert_type(
        y.reshape(n_pad, Hw, 2), jnp.int32
    ).reshape(n_pad, 8, Hw // 8)
    combine = functools.partial(
        pl.kernel,
        mesh=mesh,
        compiler_params=sc_params,
        out_type=[
            jax.ShapeDtypeStruct((T, 8, Hw // 8), jnp.int32),
            jax.ShapeDtypeStruct((T, 8, Hw // 8), jnp.int32),
        ],
        scratch_types=[
            pltpu.VMEM((2 * T // _NW,), jnp.int32),
            pltpu.VMEM((2 * T // _NW,), jnp.int32),
            pltpu.VMEM((16,), jnp.int32),
            pltpu.VMEM((_CH, 8, Hw // 8), jnp.int32),
            pltpu.VMEM((_CH, 8, Hw // 8), jnp.int32),
            pltpu.SemaphoreType.DMA,
        ],
    )(_combine_body)
    y0i, y1i = combine(y3, idx_flat, rank_flat, pad_cum16)
    y0s = lax.bitcast_convert_type(y0i.reshape(T, Hw), jnp.bfloat16)
    y1s = lax.bitcast_convert_type(y1i.reshape(T, Hw), jnp.bfloat16)

    final = (w2[:, 0:1] * y0s.reshape(T, H).astype(jnp.float32)
             + w2[:, 1:2] * y1s.reshape(T, H).astype(jnp.float32))
    return final.reshape(orig_shape)


# speedup vs baseline: 3.9194x; 3.9194x over previous
import functools

import jax
import jax.numpy as jnp
from jax import lax
from jax.experimental import pallas as pl
from jax.experimental.pallas import tpu as pltpu
from jax.experimental.pallas import tpu_sc as plsc

_BT = 256
_BR = 512
_NW = 32
_CH = 16


def _router_rank_body(x_ref, rw_ref, w_ref, idx_ref, rank_ref, counts_ref,
                      lt_ref, carry_ref):
    c = pl.program_id(0)
    bt = x_ref.shape[0]
    E = rw_ref.shape[0]

    @pl.when(c == 0)
    def _init():
        ii = lax.broadcasted_iota(jnp.int32, (bt, bt), 0)
        jj = lax.broadcasted_iota(jnp.int32, (bt, bt), 1)
        lt_ref[...] = (jj < ii).astype(jnp.float32)
        carry_ref[...] = jnp.zeros_like(carry_ref)

    x = x_ref[...]
    logits = lax.dot_general(
        x, rw_ref[...], (((1,), (1,)), ((), ())),
        preferred_element_type=jnp.float32,
    )
    s = jax.nn.sigmoid(logits)
    eio = lax.broadcasted_iota(jnp.int32, (bt, E), 1)
    m1 = jnp.max(s, axis=1, keepdims=True)
    i1 = jnp.min(jnp.where(s == m1, eio, E), axis=1, keepdims=True)
    s2 = jnp.where(eio == i1, jnp.float32(-1.0), s)
    m2 = jnp.max(s2, axis=1, keepdims=True)
    i2 = jnp.min(jnp.where(s2 == m2, eio, E), axis=1, keepdims=True)
    denom = m1 + m2 + jnp.float32(1e-20)
    w_ref[...] = jnp.concatenate([m1, m2], axis=1) / denom
    idx_ref[...] = jnp.concatenate([i1, i2], axis=1)

    oh0 = (eio == i1).astype(jnp.float32)
    oh1 = (eio == i2).astype(jnp.float32)
    both = oh0 + oh1
    pfx = lax.dot_general(
        lt_ref[...], both, (((1,), (0,)), ((), ())),
        preferred_element_type=jnp.float32,
    )
    base = carry_ref[0:1, :]
    r = pfx + base
    rank0 = jnp.sum(oh0 * r, axis=1, keepdims=True)
    rank1 = jnp.sum(oh1 * r, axis=1, keepdims=True)
    rank_ref[...] = jnp.concatenate([rank0, rank1], axis=1).astype(jnp.int32)
    newc = base + jnp.sum(both, axis=0, keepdims=True)
    carry_ref[0:1, :] = newc
    counts_ref[...] = newc.astype(jnp.int32)


def _sc_dst(idx_v, rank_v, pc_v, fo):
    ii = lax.iota(jnp.int32, 16)
    e0 = plsc.load_gather(idx_v, [fo + 2 * ii])
    e1 = plsc.load_gather(idx_v, [fo + 2 * ii + 1])
    r0 = plsc.load_gather(rank_v, [fo + 2 * ii])
    r1 = plsc.load_gather(rank_v, [fo + 2 * ii + 1])
    dst0 = plsc.load_gather(pc_v, [e0]) + r0
    dst1 = plsc.load_gather(pc_v, [e1]) + r1
    return dst0, dst1


def _dispatch_body(xbf_hbm, idx_hbm, rank_hbm, pc_hbm, xs_hbm,
                   idx_v, rank_v, pc_v, rows_v, sem):
    wid = lax.axis_index("s") * 2 + lax.axis_index("c")
    tpw = xbf_hbm.shape[0] // _NW
    base_tok = wid * tpw
    pltpu.sync_copy(pc_hbm, pc_v)
    pltpu.sync_copy(idx_hbm.at[pl.ds(base_tok * 2, tpw * 2)], idx_v)
    pltpu.sync_copy(rank_hbm.at[pl.ds(base_tok * 2, tpw * 2)], rank_v)
    for ci in range(tpw // _CH):
        tb = ci * _CH
        pltpu.sync_copy(xbf_hbm.at[pl.ds(base_tok + tb, _CH)], rows_v)
        dst0, dst1 = _sc_dst(idx_v, rank_v, pc_v, tb * 2)
        a = pltpu.async_copy(rows_v, xs_hbm.at[dst0], sem)
        b = pltpu.async_copy(rows_v, xs_hbm.at[dst1], sem)
        a.wait()
        b.wait()


def _combine_body(y_hbm, idx_hbm, rank_hbm, pc_hbm, y0_hbm, y1_hbm,
                  idx_v, rank_v, pc_v, r0_v, r1_v, sem):
    wid = lax.axis_index("s") * 2 + lax.axis_index("c")
    tpw = y0_hbm.shape[0] // _NW
    base_tok = wid * tpw
    pltpu.sync_copy(pc_hbm, pc_v)
    pltpu.sync_copy(idx_hbm.at[pl.ds(base_tok * 2, tpw * 2)], idx_v)
    pltpu.sync_copy(rank_hbm.at[pl.ds(base_tok * 2, tpw * 2)], rank_v)
    for ci in range(tpw // _CH):
        tb = ci * _CH
        dst0, dst1 = _sc_dst(idx_v, rank_v, pc_v, tb * 2)
        g0 = pltpu.async_copy(y_hbm.at[dst0], r0_v, sem)
        g1 = pltpu.async_copy(y_hbm.at[dst1], r1_v, sem)
        g0.wait()
        g1.wait()
        s0 = pltpu.async_copy(r0_v, y0_hbm.at[pl.ds(base_tok + tb, _CH)], sem)
        s1 = pltpu.async_copy(r1_v, y1_hbm.at[pl.ds(base_tok + tb, _CH)], sem)
        s0.wait()
        s1.wait()


def _moe_body(be_ref, ba_ref, xs_ref, g_ref, u_ref, d_ref, y_ref):
    b = pl.program_id(0)

    @pl.when(ba_ref[b] == 1)
    def _():
        xb = xs_ref[...]
        t1 = lax.dot_general(
            xb, g_ref[0], (((1,), (1,)), ((), ())),
            preferred_element_type=jnp.float32,
        )
        t2 = lax.dot_general(
            xb, u_ref[0], (((1,), (1,)), ((), ())),
            preferred_element_type=jnp.float32,
        )
        h = t1 * jax.nn.sigmoid(t1) * t2
        o = lax.dot_general(
            h, d_ref[0], (((1,), (1,)), ((), ())),
            preferred_element_type=jnp.float32,
        )
        y_ref[...] = o.astype(y_ref.dtype)

    @pl.when(ba_ref[b] == 0)
    def _():
        y_ref[...] = jnp.zeros_like(y_ref)


@functools.partial(jax.jit, static_argnames=())
def kernel(hidden_states, router_w, gate_w, up_w, down_w):
    orig_shape = hidden_states.shape
    H = orig_shape[-1]
    x = hidden_states.reshape(-1, H)
    T = x.shape[0]
    E, F, _ = gate_w.shape
    K = 2
    N = T * K
    n_pad = N + E * _BT
    nb = n_pad // _BT

    w2, idx2, rank2, counts = pl.pallas_call(
        _router_rank_body,
        grid=(T // _BR,),
        in_specs=[
            pl.BlockSpec((_BR, H), lambda i: (i, 0)),
            pl.BlockSpec((E, H), lambda i: (0, 0)),
        ],
        out_specs=[
            pl.BlockSpec((_BR, K), lambda i: (i, 0)),
            pl.BlockSpec((_BR, K), lambda i: (i, 0)),
            pl.BlockSpec((_BR, K), lambda i: (i, 0)),
            pl.BlockSpec((1, E), lambda i: (0, 0)),
        ],
        out_shape=[
            jax.ShapeDtypeStruct((T, K), jnp.float32),
            jax.ShapeDtypeStruct((T, K), jnp.int32),
            jax.ShapeDtypeStruct((T, K), jnp.int32),
            jax.ShapeDtypeStruct((1, E), jnp.int32),
        ],
        scratch_shapes=[
            pltpu.VMEM((_BR, _BR), jnp.float32),
            pltpu.VMEM((8, E), jnp.float32),
        ],
    )(x, router_w)

    cnt = counts[0]
    padded = ((cnt + _BT - 1) // _BT) * _BT
    pad_cum = jnp.concatenate(
        [jnp.zeros((1,), jnp.int32), jnp.cumsum(padded)[:-1].astype(jnp.int32)]
    )
    pad_cum16 = jnp.concatenate([pad_cum, jnp.zeros((16 - E,), jnp.int32)])

    block_starts = jnp.arange(nb, dtype=jnp.int32) * _BT
    cmp = (block_starts[:, None] >= pad_cum[None, :]).astype(jnp.int32)
    block_expert = jnp.sum(cmp, axis=1) - 1
    be_oh = block_expert[:, None] == jnp.arange(E)[None, :]
    pc_sel = jnp.sum(be_oh * pad_cum[None, :], axis=1).astype(jnp.int32)
    c_sel = jnp.sum(be_oh * cnt[None, :], axis=1).astype(jnp.int32)
    block_active = (block_starts - pc_sel < c_sel).astype(jnp.int32)

    idx_flat = idx2.reshape(-1)
    rank_flat = rank2.reshape(-1)

    mesh = plsc.VectorSubcoreMesh(core_axis_name="c", subcore_axis_name="s")
    sc_params = pltpu.CompilerParams(needs_layout_passes=False)
    dispatch = functools.partial(
        pl.kernel,
        mesh=mesh,
        compiler_params=sc_params,
        out_type=jax.ShapeDtypeStruct((n_pad, H), jnp.float32),
        scratch_types=[
            pltpu.VMEM((2 * T // _NW,), jnp.int32),
            pltpu.VMEM((2 * T // _NW,), jnp.int32),
            pltpu.VMEM((16,), jnp.int32),
            pltpu.VMEM((_CH, H), jnp.float32),
            pltpu.SemaphoreType.DMA,
        ],
    )(_dispatch_body)
    xs = dispatch(x, idx_flat, rank_flat, pad_cum16)

    y = pl.pallas_call(
        _moe_body,
        grid_spec=pltpu.PrefetchScalarGridSpec(
            num_scalar_prefetch=2,
            grid=(nb,),
            in_specs=[
                pl.BlockSpec((_BT, H), lambda b, be, ba: (b, 0)),
                pl.BlockSpec((1, F, H), lambda b, be, ba: (be[b], 0, 0)),
                pl.BlockSpec((1, F, H), lambda b, be, ba: (be[b], 0, 0)),
                pl.BlockSpec((1, H, F), lambda b, be, ba: (be[b], 0, 0)),
            ],
            out_specs=pl.BlockSpec((_BT, H), lambda b, be, ba: (b, 0)),
        ),
        out_shape=jax.ShapeDtypeStruct((n_pad, H), jnp.float32),
        compiler_params=pltpu.CompilerParams(
            dimension_semantics=("arbitrary",),
        ),
    )(block_expert, block_active, xs, gate_w, up_w, down_w)

    combine = functools.partial(
        pl.kernel,
        mesh=mesh,
        compiler_params=sc_params,
        out_type=[
            jax.ShapeDtypeStruct((T, H), jnp.float32),
            jax.ShapeDtypeStruct((T, H), jnp.float32),
        ],
        scratch_types=[
            pltpu.VMEM((2 * T // _NW,), jnp.int32),
            pltpu.VMEM((2 * T // _NW,), jnp.int32),
            pltpu.VMEM((16,), jnp.int32),
            pltpu.VMEM((_CH, H), jnp.float32),
            pltpu.VMEM((_CH, H), jnp.float32),
            pltpu.SemaphoreType.DMA,
        ],
    )(_combine_body)
    y0s, y1s = combine(y, idx_flat, rank_flat, pad_cum16)

    final = w2[:, 0:1] * y0s + w2[:, 1:2] * y1s
    return final.reshape(orig_shape)
